# baseline (speedup 1.0000x reference)
import functools

import numpy as np
import jax
import jax.numpy as jnp
from jax import lax
from jax.experimental import pallas as pl
from jax.experimental.pallas import tpu as pltpu
from jax.experimental.pallas import tpu_sc as plsc

H = 64
W = 64
C = 64
HW = H * W
NBR = 4
NG = 8
KK = 3
PATCH = 11
CORK = 3
SW = 76
NR = SW * SW
NRP = 5888
EXT = 6784
F2OFF = 385
BASE = 6 * SW + 6
RW0 = 384
NW = 5120
N66 = 66 * 66
SLAB66 = 4496
NEG = -3.0e38
DC = 576
DCP = 640

_f32 = jnp.float32
_highest = lax.Precision.HIGHEST
_bf16 = jnp.bfloat16


def _dot3(a, b):
  ah = a.astype(_bf16)
  al = (a - ah.astype(_f32)).astype(_bf16)
  bh = b.astype(_bf16)
  bl = (b - bh.astype(_f32)).astype(_bf16)
  r = jnp.dot(ah, bl, preferred_element_type=_f32)
  r = r + jnp.dot(al, bh, preferred_element_type=_f32)
  r = r + jnp.dot(ah, bh, preferred_element_type=_f32)
  return r


def _pad66(x, dt=None):
  c = x.shape[1]
  dt = dt or x.dtype
  x3 = x.reshape(H, W, c)
  zc = jnp.zeros((H, 1, c), dt)
  x3 = jnp.concatenate([zc, x3, zc], axis=1)
  zr = jnp.zeros((1, 66, c), dt)
  x3 = jnp.concatenate([zr, x3, zr], axis=0)
  flat = x3.reshape(N66, c)
  return jnp.concatenate([flat, jnp.zeros((SLAB66 - N66, c), dt)], axis=0)


def _conv66s(x, w33h, w33l, bias):
  cout = w33h.shape[3]
  cin = x.shape[1]
  xh = x.astype(_bf16)
  xl = (x - xh.astype(_f32)).astype(_bf16)
  sh = _pad66(xh)
  sl_ = _pad66(xl)
  out = jnp.zeros((N66, cout), _f32)
  for u in range(3):
    for v in range(3):
      o = u * 66 + v
      ah = lax.slice(sh, (o, 0), (o + N66, cin))
      al = lax.slice(sl_, (o, 0), (o + N66, cin))
      out = out + jnp.dot(ah, w33l[u, v], preferred_element_type=_f32)
      out = out + jnp.dot(al, w33h[u, v], preferred_element_type=_f32)
      out = out + jnp.dot(ah, w33h[u, v], preferred_element_type=_f32)
  return out + bias


def _extract64(x):
  c = x.shape[1]
  return x.reshape(66, 66, c)[:H, :W].reshape(HW, c)


def _l2n_sub(x):
  n = jnp.sqrt(jnp.sum(x * x, axis=0, keepdims=True))
  return x / jnp.maximum(n, 1e-12)


def _corr_body(nbrT_ref, refT_ref, u_ref, j_ref, e_scr):
  raw = jnp.transpose(
      lax.slice(nbrT_ref[...], (0, F2OFF), (C, F2OFF + NRP)))
  slab_raw = jnp.concatenate([raw, jnp.zeros((5936 - NRP, C), _f32)], axis=0)
  for ki in range(3):
    for kj in range(3):
      q = ki * 3 + kj
      off = ki * SW + kj
      u_ref[:, q * C:(q + 1) * C] = lax.slice(
          slab_raw, (off, 0), (off + NR, C))
  u_ref[:, DC:DCP] = jnp.zeros((NR, DCP - DC), _f32)

  f1 = _l2n_sub(refT_ref[...])[:, RW0:RW0 + NW]
  f2x = _l2n_sub(nbrT_ref[...])
  for d in range(PATCH * PATCH):
    ph, pw = d // PATCH, d % PATCH
    s0 = RW0 + ph * SW + pw
    sl = lax.slice(f2x, (0, s0), (C, s0 + NW))
    e = jnp.sum(f1 * sl, axis=0, keepdims=True)
    e_scr[d:d + 1, 128:128 + NW] = e

  ev = e_scr[...]
  s = jnp.zeros((128, NW), _f32)
  for du in (-1, 0, 1):
    for dv in (-1, 0, 1):
      o = 128 + du * SW + dv
      s = s + lax.slice(ev, (0, o), (128, o + NW))

  subi = lax.broadcasted_iota(jnp.int32, (128, NW), 0)
  s = jnp.where(subi < PATCH * PATCH, s, NEG)
  lane = lax.broadcasted_iota(jnp.int32, (1, NW), 1)

  js = []
  for _ in range(NBR):
    m = jnp.max(s, axis=0, keepdims=True)
    am = jnp.min(jnp.where(s == m, subi, 128), axis=0, keepdims=True)
    s = jnp.where(subi == am, NEG, s)
    dh = jnp.floor((am.astype(_f32) + 0.5) * (1.0 / PATCH)).astype(jnp.int32)
    dw = am - PATCH * dh
    j = (lane + RW0) - BASE + dh * SW + dw
    js.append(jnp.clip(j, 0, NR - 1))
  j_ref[...] = jnp.concatenate(js + js, axis=0)


def _mask_body(nbr2_ref, ref2_ref, w1h_ref, w1l_ref, b1_ref,
               w2h_ref, w2l_ref, b2_ref, wmh_ref, wml_ref, bm_ref, out_ref):
  x = jnp.concatenate([jnp.transpose(nbr2_ref[...]),
                       jnp.transpose(ref2_ref[...])], axis=1)
  h = _extract64(_conv66s(x, w1h_ref[...], w1l_ref[...], b1_ref[...]))
  h = _extract64(_conv66s(h, w2h_ref[...], w2l_ref[...], b2_ref[...]))
  m = _extract64(_conv66s(h, wmh_ref[...], wml_ref[...], bm_ref[...]))
  out_ref[...] = m


PB = 2048


def _out_body(g_ref, w2h_ref, w2l_ref, mask_ref, rm_ref, bn_ref,
              ref2_ref, wah_ref, wal_ref, ba_ref,
              out_ref, me_scr, acc_ref, l_scr):
  p = pl.program_id(0)
  k = pl.program_id(1)

  @pl.when(k == 0)
  def _():
    me_scr[...] = _dot3(mask_ref[...], rm_ref[...])

  me = me_scr[...]
  ah = lax.slice(g_ref[...], (0, 0), (PB, DC)).astype(_bf16)
  part = jnp.dot(ah, w2l_ref[0], preferred_element_type=_f32)
  part = part + jnp.dot(ah, w2h_ref[0], preferred_element_type=_f32)
  red = jnp.zeros((PB, C), _f32)
  for q in range(9):
    red = red + (lax.slice(me, (0, q * C), (PB, (q + 1) * C))
                 * lax.slice(part, (0, q * C), (PB, (q + 1) * C)))

  @pl.when(k == 0)
  def _():
    acc_ref[...] = red

  @pl.when(k > 0)
  def _():
    acc_ref[...] = acc_ref[...] + red

  @pl.when(k == NBR - 1)
  def _():
    s64 = jnp.zeros((PB, C), _f32)
    for q in range(9):
      s64 = s64 + lax.slice(me, (0, q * C), (PB, (q + 1) * C))
    lsum = acc_ref[...] + s64 * bn_ref[...]
    l_scr[pl.ds(p * PB, PB), :] = jnp.where(lsum >= 0, lsum, 0.1 * lsum)

  @pl.when((k == NBR - 1) & (p == HW // PB - 1))
  def _():
    x = jnp.concatenate([l_scr[...], jnp.transpose(ref2_ref[...])], axis=1)
    y = _extract64(_conv66s(x, wah_ref[...], wal_ref[...], ba_ref[...]))
    out_ref[...] = jnp.transpose(y)


def _sc_gather(u76, idxflat):
  mesh = plsc.VectorSubcoreMesh(core_axis_name="c", subcore_axis_name="s")
  n_idx = NBR * HW
  per_w = n_idx // 32
  chunk = 64
  n_ch = per_w // chunk

  @functools.partial(
      pl.kernel, mesh=mesh,
      out_type=jax.ShapeDtypeStruct((n_idx, DCP), _f32),
      scratch_types=[
          pltpu.VMEM((2, chunk), jnp.int32),
          pltpu.VMEM((chunk, DCP), _f32),
          pltpu.VMEM((chunk, DCP), _f32),
          pltpu.SemaphoreType.DMA,
          pltpu.SemaphoreType.DMA,
          pltpu.SemaphoreType.DMA,
          pltpu.SemaphoreType.DMA,
      ])
  def k(u_hbm, i_hbm, o_hbm, idx_v, rows0, rows1, sg0, sg1, sw0, sw1):
    wid = lax.axis_index("s") * 2 + lax.axis_index("c")
    base = wid * per_w
    rows = (rows0, rows1)
    sg = (sg0, sg1)
    sw = (sw0, sw1)

    pltpu.sync_copy(i_hbm.at[pl.ds(base, chunk)], idx_v.at[0])
    g_prev = pltpu.async_copy(u_hbm.at[idx_v.at[0]], rows0, sg0)
    w_prev = [None, None]
    for n in range(n_ch):
      b = n % 2
      nb = 1 - b
      if n + 1 < n_ch:
        pltpu.sync_copy(i_hbm.at[pl.ds(base + (n + 1) * chunk, chunk)],
                        idx_v.at[nb])
      g_prev.wait()
      if n + 1 < n_ch:
        if w_prev[nb] is not None:
          w_prev[nb].wait()
        g_prev = pltpu.async_copy(u_hbm.at[idx_v.at[nb]], rows[nb], sg[nb])
      w_prev[b] = pltpu.async_copy(
          rows[b], o_hbm.at[pl.ds(base + n * chunk, chunk)], sw[b])
    for b in range(2):
      if w_prev[b] is not None:
        w_prev[b].wait()

  return k(u76, idxflat)


def _make_rm():
  rm = np.zeros((NG * 9, DC), np.float32)
  for g in range(NG):
    for q in range(9):
      rm[g * 9 + q, q * C + g * 8:q * C + g * 8 + 8] = 1.0
  return jnp.asarray(rm)


def kernel(nbr_fea_l, ref_fea_l, w_conv1, b_conv1, w_conv2, b_conv2,
           w_mask, b_mask, w_nn, b_nn, w_agg, b_agg):
  nbr2 = nbr_fea_l.reshape(C, HW)
  ref2 = ref_fea_l.reshape(C, HW)
  nbrp = jnp.pad(nbr_fea_l[0], ((0, 0), (6, 6), (6, 6))).reshape(C, NR)
  refp = jnp.pad(ref_fea_l[0], ((0, 0), (6, 6), (6, 6))).reshape(C, NR)
  nbrT = jnp.pad(nbrp, ((0, 0), (F2OFF, EXT - NR - F2OFF)))
  refT = jnp.pad(refp, ((0, 0), (0, EXT - NR)))

  def _split(w):
    wh = w.astype(_bf16)
    return wh, (w - wh.astype(_f32)).astype(_bf16)

  w1h, w1l = _split(jnp.transpose(w_conv1, (2, 3, 1, 0)))
  w2th, w2tl = _split(jnp.transpose(w_conv2, (2, 3, 1, 0)))
  wmh, wml = _split(jnp.transpose(w_mask, (2, 3, 1, 0)))
  wah, wal = _split(jnp.transpose(w_agg, (2, 3, 1, 0)))
  b1r = b_conv1[None, :]
  b2r = b_conv2[None, :]
  bmr = b_mask[None, :]
  bar = b_agg[None, :]

  w5 = w_nn.reshape(C, NBR, C, 3, 3)
  wp = jnp.pad(w5, ((0, 0), (0, 0), (0, 0), (2, 2), (2, 2)))
  blocks = [wp[:, :, :, 3 - oy:6 - oy, 3 - ox:6 - ox]
            for oy in range(3) for ox in range(3)]
  a = jnp.stack(blocks, axis=0)
  w2b = jnp.transpose(a, (2, 4, 5, 3, 0, 1)).reshape(NBR, DC, DC)
  w2h, w2l = _split(w2b)
  bnr = b_nn[None, :]
  rm = _make_rm()

  u76, jraw = pl.pallas_call(
      _corr_body,
      out_shape=(jax.ShapeDtypeStruct((NR, DCP), _f32),
                 jax.ShapeDtypeStruct((8, NW), jnp.int32)),
      scratch_shapes=[pltpu.VMEM((128, 256 + NW), _f32)],
  )(nbrT, refT)

  jfull = jnp.pad(jraw[:NBR], ((0, 0), (RW0, NR - RW0 - NW)))
  idxflat = (jfull.reshape(NBR, SW, SW)[:, 6:6 + H, 6:6 + W]
             .reshape(NBR * HW))

  mask_m = pl.pallas_call(
      _mask_body,
      out_shape=jax.ShapeDtypeStruct((HW, NG * 9), _f32),
  )(nbr2, ref2, w1h, w1l, b1r, w2th, w2tl, b2r, wmh, wml, bmr)

  g = _sc_gather(u76, idxflat)

  npb = HW // PB
  out_c = pl.pallas_call(
      _out_body,
      grid=(npb, NBR),
      in_specs=[
          pl.BlockSpec((PB, DCP), lambda p, k: (k * npb + p, 0)),
          pl.BlockSpec((1, DC, DC), lambda p, k: (k, 0, 0)),
          pl.BlockSpec((1, DC, DC), lambda p, k: (k, 0, 0)),
          pl.BlockSpec((PB, NG * 9), lambda p, k: (p, 0)),
          pl.BlockSpec((NG * 9, DC), lambda p, k: (0, 0)),
          pl.BlockSpec((1, C), lambda p, k: (0, 0)),
          pl.BlockSpec((C, HW), lambda p, k: (0, 0)),
          pl.BlockSpec((3, 3, 2 * C, C), lambda p, k: (0, 0, 0, 0)),
          pl.BlockSpec((3, 3, 2 * C, C), lambda p, k: (0, 0, 0, 0)),
          pl.BlockSpec((1, C), lambda p, k: (0, 0)),
      ],
      out_specs=pl.BlockSpec((C, HW), lambda p, k: (0, 0)),
      out_shape=jax.ShapeDtypeStruct((C, HW), _f32),
      scratch_shapes=[pltpu.VMEM((PB, DC), _f32),
                      pltpu.VMEM((PB, C), _f32),
                      pltpu.VMEM((HW, C), _f32)],
  )(g, w2h, w2l, mask_m, rm, bnr, ref2, wah, wal, bar)

  return out_c.reshape(1, C, H, W)


# speedup vs baseline: 1.0601x; 1.0601x over previous
import functools

import numpy as np
import jax
import jax.numpy as jnp
from jax import lax
from jax.experimental import pallas as pl
from jax.experimental.pallas import tpu as pltpu
from jax.experimental.pallas import tpu_sc as plsc

H = 64
W = 64
C = 64
HW = H * W
NBR = 4
NG = 8
KK = 3
PATCH = 11
CORK = 3
SW = 76
NR = SW * SW
NRP = 5888
EXT = 6784
F2OFF = 385
BASE = 6 * SW + 6
RW0 = 384
NW = 5120
N66 = 66 * 66
SLAB66 = 4496
NEG = -3.0e38
DC = 576
DCP = 640

_f32 = jnp.float32
_highest = lax.Precision.HIGHEST
_bf16 = jnp.bfloat16


def _dot3(a, b):
  ah = a.astype(_bf16)
  al = (a - ah.astype(_f32)).astype(_bf16)
  bh = b.astype(_bf16)
  bl = (b - bh.astype(_f32)).astype(_bf16)
  r = jnp.dot(ah, bl, preferred_element_type=_f32)
  r = r + jnp.dot(al, bh, preferred_element_type=_f32)
  r = r + jnp.dot(ah, bh, preferred_element_type=_f32)
  return r


def _pad66(x, dt=None):
  c = x.shape[1]
  dt = dt or x.dtype
  x3 = x.reshape(H, W, c)
  zc = jnp.zeros((H, 1, c), dt)
  x3 = jnp.concatenate([zc, x3, zc], axis=1)
  zr = jnp.zeros((1, 66, c), dt)
  x3 = jnp.concatenate([zr, x3, zr], axis=0)
  flat = x3.reshape(N66, c)
  return jnp.concatenate([flat, jnp.zeros((SLAB66 - N66, c), dt)], axis=0)


def _conv66s(x, w33h, w33l, bias):
  cout = w33h.shape[3]
  cin = x.shape[1]
  xh = x.astype(_bf16)
  xl = (x - xh.astype(_f32)).astype(_bf16)
  sh = _pad66(xh)
  sl_ = _pad66(xl)
  out = jnp.zeros((N66, cout), _f32)
  for u in range(3):
    for v in range(3):
      o = u * 66 + v
      ah = lax.slice(sh, (o, 0), (o + N66, cin))
      al = lax.slice(sl_, (o, 0), (o + N66, cin))
      out = out + jnp.dot(ah, w33l[u, v], preferred_element_type=_f32)
      out = out + jnp.dot(al, w33h[u, v], preferred_element_type=_f32)
      out = out + jnp.dot(ah, w33h[u, v], preferred_element_type=_f32)
  return out + bias


def _extract64(x):
  c = x.shape[1]
  return x.reshape(66, 66, c)[:H, :W].reshape(HW, c)


def _l2n_sub(x):
  n = jnp.sqrt(jnp.sum(x * x, axis=0, keepdims=True))
  return x / jnp.maximum(n, 1e-12)


def _corr_body(nbrT_ref, refT_ref, u_ref, j_ref, e_scr):
  raw = jnp.transpose(
      lax.slice(nbrT_ref[...], (0, F2OFF), (C, F2OFF + NRP)))
  slab_raw = jnp.concatenate([raw, jnp.zeros((5936 - NRP, C), _f32)], axis=0)
  for ki in range(3):
    for kj in range(3):
      q = ki * 3 + kj
      off = ki * SW + kj
      u_ref[:, q * C:(q + 1) * C] = lax.slice(
          slab_raw, (off, 0), (off + NR, C))
  u_ref[:, DC:DCP] = jnp.zeros((NR, DCP - DC), _f32)

  f1 = _l2n_sub(refT_ref[...])[:, RW0:RW0 + NW]
  f2x = _l2n_sub(nbrT_ref[...])
  for d in range(PATCH * PATCH):
    ph, pw = d // PATCH, d % PATCH
    s0 = RW0 + ph * SW + pw
    sl = lax.slice(f2x, (0, s0), (C, s0 + NW))
    e = jnp.sum(f1 * sl, axis=0, keepdims=True)
    e_scr[d:d + 1, 128:128 + NW] = e

  ev = e_scr[...]
  s = jnp.zeros((128, NW), _f32)
  for du in (-1, 0, 1):
    for dv in (-1, 0, 1):
      o = 128 + du * SW + dv
      s = s + lax.slice(ev, (0, o), (128, o + NW))

  subi = lax.broadcasted_iota(jnp.int32, (128, NW), 0)
  s = jnp.where(subi < PATCH * PATCH, s, NEG)
  lane = lax.broadcasted_iota(jnp.int32, (1, NW), 1)

  js = []
  for _ in range(NBR):
    m = jnp.max(s, axis=0, keepdims=True)
    am = jnp.min(jnp.where(s == m, subi, 128), axis=0, keepdims=True)
    s = jnp.where(subi == am, NEG, s)
    dh = jnp.floor((am.astype(_f32) + 0.5) * (1.0 / PATCH)).astype(jnp.int32)
    dw = am - PATCH * dh
    j = (lane + RW0) - BASE + dh * SW + dw
    js.append(jnp.clip(j, 0, NR - 1))
  j_ref[...] = jnp.concatenate(js + js, axis=0)


def _mask_body(nbr2_ref, ref2_ref, w1h_ref, w1l_ref, b1_ref,
               w2h_ref, w2l_ref, b2_ref, wmh_ref, wml_ref, bm_ref, out_ref):
  x = jnp.concatenate([jnp.transpose(nbr2_ref[...]),
                       jnp.transpose(ref2_ref[...])], axis=1)
  h = _extract64(_conv66s(x, w1h_ref[...], w1l_ref[...], b1_ref[...]))
  h = _extract64(_conv66s(h, w2h_ref[...], w2l_ref[...], b2_ref[...]))
  m = _extract64(_conv66s(h, wmh_ref[...], wml_ref[...], bm_ref[...]))
  out_ref[...] = m


PB = 1024


def _out_body(g_ref, w2h_ref, w2l_ref, mask_ref, rm_ref, bn_ref,
              ref2_ref, wah_ref, wal_ref, ba_ref,
              out_ref, me_scr, acc_ref, l_scr):
  p = pl.program_id(0)
  k = pl.program_id(1)

  @pl.when(k == 0)
  def _():
    me_scr[...] = _dot3(mask_ref[...], rm_ref[...])

  me = me_scr[...]
  ah = lax.slice(g_ref[...], (0, 0), (PB, DC)).astype(_bf16)
  part = jnp.dot(ah, w2l_ref[0], preferred_element_type=_f32)
  part = part + jnp.dot(ah, w2h_ref[0], preferred_element_type=_f32)
  red = jnp.zeros((PB, C), _f32)
  for q in range(9):
    red = red + (lax.slice(me, (0, q * C), (PB, (q + 1) * C))
                 * lax.slice(part, (0, q * C), (PB, (q + 1) * C)))

  @pl.when(k == 0)
  def _():
    acc_ref[...] = red

  @pl.when(k > 0)
  def _():
    acc_ref[...] = acc_ref[...] + red

  @pl.when(k == NBR - 1)
  def _():
    s64 = jnp.zeros((PB, C), _f32)
    for q in range(9):
      s64 = s64 + lax.slice(me, (0, q * C), (PB, (q + 1) * C))
    lsum = acc_ref[...] + s64 * bn_ref[...]
    l_scr[pl.ds(p * PB, PB), :] = jnp.where(lsum >= 0, lsum, 0.1 * lsum)

  @pl.when((k == NBR - 1) & (p == HW // PB - 1))
  def _():
    x = jnp.concatenate([l_scr[...], jnp.transpose(ref2_ref[...])], axis=1)
    y = _extract64(_conv66s(x, wah_ref[...], wal_ref[...], ba_ref[...]))
    out_ref[...] = jnp.transpose(y)


def _sc_gather(u76, idxflat):
  mesh = plsc.VectorSubcoreMesh(core_axis_name="c", subcore_axis_name="s")
  n_idx = NBR * HW
  per_w = n_idx // 32
  chunk = 64
  n_ch = per_w // chunk

  @functools.partial(
      pl.kernel, mesh=mesh,
      out_type=jax.ShapeDtypeStruct((n_idx, DCP), _f32),
      scratch_types=[
          pltpu.VMEM((2, chunk), jnp.int32),
          pltpu.VMEM((chunk, DCP), _f32),
          pltpu.VMEM((chunk, DCP), _f32),
          pltpu.SemaphoreType.DMA,
          pltpu.SemaphoreType.DMA,
          pltpu.SemaphoreType.DMA,
          pltpu.SemaphoreType.DMA,
      ])
  def k(u_hbm, i_hbm, o_hbm, idx_v, rows0, rows1, sg0, sg1, sw0, sw1):
    wid = lax.axis_index("s") * 2 + lax.axis_index("c")
    base = wid * per_w
    rows = (rows0, rows1)
    sg = (sg0, sg1)
    sw = (sw0, sw1)

    pltpu.sync_copy(i_hbm.at[pl.ds(base, chunk)], idx_v.at[0])
    g_prev = pltpu.async_copy(u_hbm.at[idx_v.at[0]], rows0, sg0)
    w_prev = [None, None]
    for n in range(n_ch):
      b = n % 2
      nb = 1 - b
      if n + 1 < n_ch:
        pltpu.sync_copy(i_hbm.at[pl.ds(base + (n + 1) * chunk, chunk)],
                        idx_v.at[nb])
      g_prev.wait()
      if n + 1 < n_ch:
        if w_prev[nb] is not None:
          w_prev[nb].wait()
        g_prev = pltpu.async_copy(u_hbm.at[idx_v.at[nb]], rows[nb], sg[nb])
      w_prev[b] = pltpu.async_copy(
          rows[b], o_hbm.at[pl.ds(base + n * chunk, chunk)], sw[b])
    for b in range(2):
      if w_prev[b] is not None:
        w_prev[b].wait()

  return k(u76, idxflat)


def _make_rm():
  rm = np.zeros((NG * 9, DC), np.float32)
  for g in range(NG):
    for q in range(9):
      rm[g * 9 + q, q * C + g * 8:q * C + g * 8 + 8] = 1.0
  return jnp.asarray(rm)


def kernel(nbr_fea_l, ref_fea_l, w_conv1, b_conv1, w_conv2, b_conv2,
           w_mask, b_mask, w_nn, b_nn, w_agg, b_agg):
  nbr2 = nbr_fea_l.reshape(C, HW)
  ref2 = ref_fea_l.reshape(C, HW)
  nbrp = jnp.pad(nbr_fea_l[0], ((0, 0), (6, 6), (6, 6))).reshape(C, NR)
  refp = jnp.pad(ref_fea_l[0], ((0, 0), (6, 6), (6, 6))).reshape(C, NR)
  nbrT = jnp.pad(nbrp, ((0, 0), (F2OFF, EXT - NR - F2OFF)))
  refT = jnp.pad(refp, ((0, 0), (0, EXT - NR)))

  def _split(w):
    wh = w.astype(_bf16)
    return wh, (w - wh.astype(_f32)).astype(_bf16)

  w1h, w1l = _split(jnp.transpose(w_conv1, (2, 3, 1, 0)))
  w2th, w2tl = _split(jnp.transpose(w_conv2, (2, 3, 1, 0)))
  wmh, wml = _split(jnp.transpose(w_mask, (2, 3, 1, 0)))
  wah, wal = _split(jnp.transpose(w_agg, (2, 3, 1, 0)))
  b1r = b_conv1[None, :]
  b2r = b_conv2[None, :]
  bmr = b_mask[None, :]
  bar = b_agg[None, :]

  w5 = w_nn.reshape(C, NBR, C, 3, 3)
  wp = jnp.pad(w5, ((0, 0), (0, 0), (0, 0), (2, 2), (2, 2)))
  blocks = [wp[:, :, :, 3 - oy:6 - oy, 3 - ox:6 - ox]
            for oy in range(3) for ox in range(3)]
  a = jnp.stack(blocks, axis=0)
  w2b = jnp.transpose(a, (2, 4, 5, 3, 0, 1)).reshape(NBR, DC, DC)
  w2h, w2l = _split(w2b)
  bnr = b_nn[None, :]
  rm = _make_rm()

  u76, jraw = pl.pallas_call(
      _corr_body,
      out_shape=(jax.ShapeDtypeStruct((NR, DCP), _f32),
                 jax.ShapeDtypeStruct((8, NW), jnp.int32)),
      scratch_shapes=[pltpu.VMEM((128, 256 + NW), _f32)],
  )(nbrT, refT)

  jfull = jnp.pad(jraw[:NBR], ((0, 0), (RW0, NR - RW0 - NW)))
  idxflat = (jfull.reshape(NBR, SW, SW)[:, 6:6 + H, 6:6 + W]
             .reshape(NBR * HW))

  mask_m = pl.pallas_call(
      _mask_body,
      out_shape=jax.ShapeDtypeStruct((HW, NG * 9), _f32),
  )(nbr2, ref2, w1h, w1l, b1r, w2th, w2tl, b2r, wmh, wml, bmr)

  g = _sc_gather(u76, idxflat)

  npb = HW // PB
  out_c = pl.pallas_call(
      _out_body,
      grid=(npb, NBR),
      in_specs=[
          pl.BlockSpec((PB, DCP), lambda p, k: (k * npb + p, 0)),
          pl.BlockSpec((1, DC, DC), lambda p, k: (k, 0, 0)),
          pl.BlockSpec((1, DC, DC), lambda p, k: (k, 0, 0)),
          pl.BlockSpec((PB, NG * 9), lambda p, k: (p, 0)),
          pl.BlockSpec((NG * 9, DC), lambda p, k: (0, 0)),
          pl.BlockSpec((1, C), lambda p, k: (0, 0)),
          pl.BlockSpec((C, HW), lambda p, k: (0, 0)),
          pl.BlockSpec((3, 3, 2 * C, C), lambda p, k: (0, 0, 0, 0)),
          pl.BlockSpec((3, 3, 2 * C, C), lambda p, k: (0, 0, 0, 0)),
          pl.BlockSpec((1, C), lambda p, k: (0, 0)),
      ],
      out_specs=pl.BlockSpec((C, HW), lambda p, k: (0, 0)),
      out_shape=jax.ShapeDtypeStruct((C, HW), _f32),
      scratch_shapes=[pltpu.VMEM((PB, DC), _f32),
                      pltpu.VMEM((PB, C), _f32),
                      pltpu.VMEM((HW, C), _f32)],
  )(g, w2h, w2l, mask_m, rm, bnr, ref2, wah, wal, bar)

  return out_c.reshape(1, C, H, W)
